# Initial kernel scaffold; baseline (speedup 1.0000x reference)
import jax
import jax.numpy as jnp
from jax.experimental import pallas as pl


def kernel(tok_ids, embedding):
    raise NotImplementedError("write your pallas kernel here")


# speedup vs baseline: 1.4768x; 1.4768x over previous
import functools

import jax
import jax.numpy as jnp
from jax import lax
from jax.experimental import pallas as pl
from jax.experimental.pallas import tpu as pltpu
from jax.experimental.pallas import tpu_sc as plsc

N_VOCAB = 1000000
D_MODEL = 32
BATCH = 4096
HIST = 200

N_ROWS = BATCH * HIST
NC = 2
NS = 16
NW = NC * NS
BPW = N_ROWS // NW
IDX_MINOR = 128
N_SLICES = BPW // IDX_MINOR
CHUNK_SLICES = 8
CHUNK_ROWS = CHUNK_SLICES * IDX_MINOR
N_CHUNKS = N_SLICES // CHUNK_SLICES

_mesh = plsc.VectorSubcoreMesh(core_axis_name="c", subcore_axis_name="s")


@functools.partial(
    pl.kernel,
    mesh=_mesh,
    out_type=jax.ShapeDtypeStruct((N_ROWS, D_MODEL), jnp.float32),
    scratch_types=[
        pltpu.VMEM((N_SLICES, IDX_MINOR), jnp.int32),
        pltpu.VMEM((CHUNK_ROWS, D_MODEL), jnp.float32),
        pltpu.SemaphoreType.DMA,
    ],
    compiler_params=pltpu.CompilerParams(use_tc_tiling_on_sc=False),
)
def _embed_gather(table_hbm, idx_hbm, out_hbm, idx_v, rows_v, gsem):
    wid = lax.axis_index("s") * NC + lax.axis_index("c")
    base = wid * BPW
    pltpu.sync_copy(idx_hbm.at[wid], idx_v)

    def chunk_body(c, carry):
        copies = []
        for k in range(CHUNK_SLICES):
            copies.append(
                pltpu.async_copy(
                    table_hbm.at[idx_v.at[c * CHUNK_SLICES + k]],
                    rows_v.at[pl.ds(k * IDX_MINOR, IDX_MINOR)],
                    gsem,
                )
            )
        for cp in copies:
            cp.wait()
        pltpu.sync_copy(rows_v, out_hbm.at[pl.ds(base + c * CHUNK_ROWS, CHUNK_ROWS)])
        return carry

    lax.fori_loop(0, N_CHUNKS, chunk_body, 0)


def kernel(tok_ids, embedding):
    idx = tok_ids.astype(jnp.int32).reshape(NW, N_SLICES, IDX_MINOR)
    out = _embed_gather(embedding, idx)
    return out.reshape(BATCH, HIST, D_MODEL)


# speedup vs baseline: 1.5005x; 1.0161x over previous
import functools

import jax
import jax.numpy as jnp
from jax import lax
from jax.experimental import pallas as pl
from jax.experimental.pallas import tpu as pltpu
from jax.experimental.pallas import tpu_sc as plsc

N_VOCAB = 1000000
D_MODEL = 32
BATCH = 4096
HIST = 200

N_ROWS = BATCH * HIST
NC = 2
NS = 16
NW = NC * NS
BPW = N_ROWS // NW
IDX_MINOR = 128
N_SLICES = BPW // IDX_MINOR
CHUNK_SLICES = 10
CHUNK_ROWS = CHUNK_SLICES * IDX_MINOR
N_CHUNKS = N_SLICES // CHUNK_SLICES

_mesh = plsc.VectorSubcoreMesh(core_axis_name="c", subcore_axis_name="s")


@functools.partial(
    pl.kernel,
    mesh=_mesh,
    out_type=jax.ShapeDtypeStruct((N_ROWS, D_MODEL), jnp.float32),
    scratch_types=[
        pltpu.VMEM((N_SLICES, IDX_MINOR), jnp.int32),
        pltpu.VMEM((2, CHUNK_ROWS, D_MODEL), jnp.float32),
        pltpu.SemaphoreType.DMA,
        pltpu.SemaphoreType.DMA,
        pltpu.SemaphoreType.DMA,
        pltpu.SemaphoreType.DMA,
    ],
    compiler_params=pltpu.CompilerParams(use_tc_tiling_on_sc=False),
)
def _embed_gather(table_hbm, idx_hbm, out_hbm, idx_v, rows_v,
                  gsem0, gsem1, wsem0, wsem1):
    wid = lax.axis_index("s") * NC + lax.axis_index("c")
    base = wid * BPW
    gsems = (gsem0, gsem1)
    wsems = (wsem0, wsem1)

    pltpu.sync_copy(idx_hbm.at[wid], idx_v)

    def issue_gathers(c, b):
        for k in range(CHUNK_SLICES):
            pltpu.async_copy(
                table_hbm.at[idx_v.at[c * CHUNK_SLICES + k]],
                rows_v.at[b, pl.ds(k * IDX_MINOR, IDX_MINOR)],
                gsems[b],
            )

    def wait_gathers(b):
        for k in range(CHUNK_SLICES):
            pltpu.make_async_copy(
                table_hbm.at[pl.ds(0, IDX_MINOR)],
                rows_v.at[b, pl.ds(k * IDX_MINOR, IDX_MINOR)],
                gsems[b],
            ).wait()

    def issue_write(c, b):
        pltpu.async_copy(
            rows_v.at[b],
            out_hbm.at[pl.ds(base + c * CHUNK_ROWS, CHUNK_ROWS)],
            wsems[b],
        )

    def wait_write(b):
        pltpu.make_async_copy(
            rows_v.at[b],
            out_hbm.at[pl.ds(base, CHUNK_ROWS)],
            wsems[b],
        ).wait()

    issue_gathers(0, 0)
    issue_gathers(1, 1)
    wait_gathers(0)
    issue_write(0, 0)

    def pair_body(p, carry):
        for b in range(2):
            c = 2 * p + b
            wait_write(b)
            issue_gathers(c, b)
            wait_gathers(1 - b)
            issue_write(c - 1, 1 - b)
        return carry

    lax.fori_loop(1, N_CHUNKS // 2, pair_body, 0)

    wait_gathers(1)
    issue_write(N_CHUNKS - 1, 1)
    wait_write(0)
    wait_write(1)


def kernel(tok_ids, embedding):
    idx = tok_ids.astype(jnp.int32).reshape(NW, N_SLICES, IDX_MINOR)
    out = _embed_gather(embedding, idx)
    return out.reshape(BATCH, HIST, D_MODEL)
